# Initial kernel scaffold; baseline (speedup 1.0000x reference)
import jax
import jax.numpy as jnp
from jax.experimental import pallas as pl


def kernel(x, edge_index, W1, b1, W2, b2):
    raise NotImplementedError("write your pallas kernel here")


# speedup vs baseline: 10.9539x; 10.9539x over previous
import functools

import jax
import jax.numpy as jnp
from jax import lax
from jax.experimental import pallas as pl
from jax.experimental.pallas import tpu as pltpu
from jax.experimental.pallas import tpu_sc as plsc

NC = 2
NS = 16
NW = NC * NS
CH = 80
DW = 16


@functools.lru_cache(maxsize=None)
def _make_sc_scatter(N, F, Ep, with_deg):
    EPW = Ep // NW
    steps = EPW // CH
    Nz = ((N + NS * 8 - 1) // (NS * 8)) * (NS * 8)
    NPS = Nz // NS
    Nacc = Nz + 16

    mesh = plsc.VectorSubcoreMesh(
        core_axis_name="c", subcore_axis_name="s", num_cores=NC, num_subcores=NS
    )

    out_type = [jax.ShapeDtypeStruct((NC, Nz, F), jnp.float32)]
    scratch = [
        pltpu.VMEM((CH,), jnp.int32),
        pltpu.VMEM((CH,), jnp.int32),
        pltpu.VMEM((CH, F), jnp.float32),
        pltpu.VMEM_SHARED((Nacc, F), jnp.float32),
        pltpu.SemaphoreType.DMA,
    ]
    if with_deg:
        out_type.append(jax.ShapeDtypeStruct((NC, Nz, DW), jnp.float32))
        scratch += [
            pltpu.VMEM((CH, DW), jnp.float32),
            pltpu.VMEM_SHARED((Nacc, DW), jnp.float32),
        ]

    def body(row_hbm, col_hbm, h_hbm, zf_hbm, zd_hbm, e1_hbm,
             p_hbm, *rest):
        if with_deg:
            dp_hbm, idx_row, idx_col, rows, acc, sem, ones_v, dacc = rest
        else:
            idx_row, idx_col, rows, acc, sem = rest
        cid = lax.axis_index("c")
        sid = lax.axis_index("s")
        wid = cid * NS + sid

        pltpu.sync_copy(zf_hbm, acc.at[pl.ds(sid * NPS, NPS)])
        if with_deg:
            pltpu.sync_copy(zd_hbm, dacc.at[pl.ds(sid * NPS, NPS)])
            pltpu.sync_copy(e1_hbm, ones_v)
        plsc.subcore_barrier()

        def step(i, carry):
            off = wid * EPW + i * CH
            pltpu.sync_copy(row_hbm.at[pl.ds(off, CH)], idx_row)
            pltpu.sync_copy(col_hbm.at[pl.ds(off, CH)], idx_col)
            pltpu.async_copy(h_hbm.at[idx_col], rows, sem).wait()
            pltpu.sync_copy(rows, acc.at[idx_row], add=True)
            if with_deg:
                pltpu.sync_copy(ones_v, dacc.at[idx_row], add=True)
            return carry

        lax.fori_loop(0, steps, step, 0)
        plsc.subcore_barrier()

        sl = pl.ds(sid * NPS, NPS)
        pltpu.sync_copy(acc.at[sl], p_hbm.at[cid].at[sl])
        if with_deg:
            pltpu.sync_copy(dacc.at[sl], dp_hbm.at[cid].at[sl])

    return pl.kernel(body, out_type=tuple(out_type), mesh=mesh,
                     scratch_types=tuple(scratch),
                     compiler_params=pltpu.CompilerParams(
                         use_tc_tiling_on_sc=False))


def _sc_scatter(row, col, h, with_deg):
    N, F = h.shape
    E = row.shape[0]
    Nz = ((N + NS * 8 - 1) // (NS * 8)) * (NS * 8)
    Ep = ((E + NW * CH - 1) // (NW * CH)) * (NW * CH)
    if Ep != E:
        row = jnp.concatenate([row, jnp.full((Ep - E,), Nz, jnp.int32)])
        col = jnp.concatenate([col, jnp.zeros((Ep - E,), jnp.int32)])
    NPS = Nz // NS
    zf = jnp.zeros((NPS, F), jnp.float32)
    zd = jnp.zeros((NPS, DW), jnp.float32)
    e1 = jnp.zeros((CH, DW), jnp.float32).at[:, 0].set(1.0)
    k = _make_sc_scatter(N, F, Ep, with_deg)
    return k(row, col, h, zf, zd, e1)


def _mm_body(x_ref, w_ref, o_ref):
    o_ref[...] = jnp.dot(x_ref[...], w_ref[...],
                         preferred_element_type=jnp.float32)


def _mid_body(p_ref, h1_ref, dp_ref, b1_ref, w2_ref, o_ref):
    deg = dp_ref[0, :, 0:1] + dp_ref[1, :, 0:1] + 1.0
    s = (p_ref[0] + p_ref[1] + h1_ref[...]) * (1.0 / deg) + b1_ref[...]
    h = jnp.maximum(s, 0.0)
    o_ref[...] = jnp.dot(h, w2_ref[...], preferred_element_type=jnp.float32)


def _out_body(q_ref, h2_ref, dp_ref, b2_ref, o_ref):
    deg = dp_ref[0, :, 0:1] + dp_ref[1, :, 0:1] + 1.0
    o_ref[...] = (q_ref[0] + q_ref[1] + h2_ref[...]) * (1.0 / deg) \
        + b2_ref[...]


def _row_block(N):
    for r in (1000, 500, 250, 200, 125, 100, 50, 40, 25, 20, 10, 8, 5, 4, 2):
        if N % r == 0:
            return r
    return N


def kernel(x, edge_index, W1, b1, W2, b2):
    N, NF = x.shape
    NH = W1.shape[1]
    F2 = W2.shape[1]
    row = edge_index[0]
    col = edge_index[1]
    R = _row_block(N)
    G = N // R

    h1 = pl.pallas_call(
        _mm_body,
        grid=(G,),
        in_specs=[pl.BlockSpec((R, NF), lambda i: (i, 0)),
                  pl.BlockSpec((NF, NH), lambda i: (0, 0))],
        out_specs=pl.BlockSpec((R, NH), lambda i: (i, 0)),
        out_shape=jax.ShapeDtypeStruct((N, NH), jnp.float32),
    )(x, W1)

    p1, dp = _sc_scatter(row, col, h1, with_deg=True)

    h2 = pl.pallas_call(
        _mid_body,
        grid=(G,),
        in_specs=[pl.BlockSpec((NC, R, NH), lambda i: (0, i, 0)),
                  pl.BlockSpec((R, NH), lambda i: (i, 0)),
                  pl.BlockSpec((NC, R, DW), lambda i: (0, i, 0)),
                  pl.BlockSpec((1, NH), lambda i: (0, 0)),
                  pl.BlockSpec((NH, F2), lambda i: (0, 0))],
        out_specs=pl.BlockSpec((R, F2), lambda i: (i, 0)),
        out_shape=jax.ShapeDtypeStruct((N, F2), jnp.float32),
    )(p1[:, :N], h1, dp[:, :N], b1.reshape(1, NH), W2)

    (p2,) = _sc_scatter(row, col, h2, with_deg=False)

    out = pl.pallas_call(
        _out_body,
        grid=(G,),
        in_specs=[pl.BlockSpec((NC, R, F2), lambda i: (0, i, 0)),
                  pl.BlockSpec((R, F2), lambda i: (i, 0)),
                  pl.BlockSpec((NC, R, DW), lambda i: (0, i, 0)),
                  pl.BlockSpec((1, F2), lambda i: (0, 0))],
        out_specs=pl.BlockSpec((R, F2), lambda i: (i, 0)),
        out_shape=jax.ShapeDtypeStruct((N, F2), jnp.float32),
    )(p2[:, :N], h2, dp[:, :N], b2.reshape(1, F2))

    return out
